# Initial kernel scaffold; baseline (speedup 1.0000x reference)
import jax
import jax.numpy as jnp
from jax.experimental import pallas as pl


def kernel(x, W_conv, gamma, beta, W1, b1, W2, b2):
    raise NotImplementedError("write your pallas kernel here")


# speedup vs baseline: 6.9017x; 6.9017x over previous
import functools

import jax
import jax.numpy as jnp
from jax import lax
from jax.experimental import pallas as pl
from jax.experimental.pallas import tpu as pltpu
from jax.experimental.pallas import tpu_sc as plsc

BB, CC, NN, KK = 8, 256, 2048, 16
CO = 256
TN = 256
NW = 32
PPW = BB * NN // NW
CHP = 8


def _prep_body(x_ref, wa_ref, wd_ref, pt_ref, qt_ref, sq_ref):
    xb = x_ref[0]
    pt_ref[0] = lax.dot_general(xb, wa_ref[...], (((0,), (1,)), ((), ())),
                                preferred_element_type=jnp.float32)
    qt_ref[0] = lax.dot_general(xb, wd_ref[...], (((0,), (1,)), ((), ())),
                                preferred_element_type=jnp.float32)
    sq_ref[0, 0] = jnp.sum(xb * xb, axis=0)


def _prep(x, wa, wd):
    return pl.pallas_call(
        _prep_body,
        grid=(BB,),
        in_specs=[
            pl.BlockSpec((1, CC, NN), lambda b: (b, 0, 0)),
            pl.BlockSpec((CO, CC), lambda b: (0, 0)),
            pl.BlockSpec((CO, CC), lambda b: (0, 0)),
        ],
        out_specs=[
            pl.BlockSpec((1, NN, CO), lambda b: (b, 0, 0)),
            pl.BlockSpec((1, NN, CO), lambda b: (b, 0, 0)),
            pl.BlockSpec((1, 1, NN), lambda b: (b, 0, 0)),
        ],
        out_shape=[
            jax.ShapeDtypeStruct((BB, NN, CO), jnp.float32),
            jax.ShapeDtypeStruct((BB, NN, CO), jnp.float32),
            jax.ShapeDtypeStruct((BB, 1, NN), jnp.float32),
        ],
    )(x, wa, wd)


def _knn_body(xrow_ref, xall_ref, sq_ref, sqrow_ref, idx_ref):
    b = pl.program_id(0)
    xr = xrow_ref[0]
    xa = xall_ref[0]
    inner = lax.dot_general(xr, xa, (((0,), (0,)), ((), ())),
                            preferred_element_type=jnp.float32)
    sq = sq_ref[0]
    sqr = sqrow_ref[0]
    d = jnp.transpose(sqr) + sq - 2.0 * inner
    col_ids = lax.broadcasted_iota(jnp.int32, (TN, NN), 1)
    cols = []
    for _ in range(KK):
        dmin = jnp.min(d, axis=1, keepdims=True)
        am = jnp.min(jnp.where(d == dmin, col_ids, NN), axis=1,
                     keepdims=True)
        cols.append(am)
        d = jnp.where(col_ids == am, jnp.inf, d)
    idx_ref[0] = jnp.concatenate(cols, axis=1) + b * NN


def _knn(x, sq):
    return pl.pallas_call(
        _knn_body,
        grid=(BB, NN // TN),
        in_specs=[
            pl.BlockSpec((1, CC, TN), lambda b, j: (b, 0, j)),
            pl.BlockSpec((1, CC, NN), lambda b, j: (b, 0, 0)),
            pl.BlockSpec((1, 1, NN), lambda b, j: (b, 0, 0)),
            pl.BlockSpec((1, 1, TN), lambda b, j: (b, 0, j)),
        ],
        out_specs=pl.BlockSpec((1, TN, KK), lambda b, j: (b, j, 0)),
        out_shape=jax.ShapeDtypeStruct((BB, NN, KK), jnp.int32),
    )(x, x, sq, sq)


def _sc_gather_reduce(pt_flat, idx_flat):
    mesh = plsc.VectorSubcoreMesh(core_axis_name="c", subcore_axis_name="s")

    @functools.partial(
        pl.kernel,
        mesh=mesh,
        out_type=(
            jax.ShapeDtypeStruct((BB * NN, CO), jnp.float32),
            jax.ShapeDtypeStruct((BB * NN, CO), jnp.float32),
            jax.ShapeDtypeStruct((BB * NN, CO), jnp.float32),
        ),
        scratch_types=[
            pltpu.VMEM((CHP * KK,), jnp.int32),
            pltpu.VMEM((CHP * KK, CO), jnp.float32),
            pltpu.VMEM((CHP, CO), jnp.float32),
            pltpu.VMEM((CHP, CO), jnp.float32),
            pltpu.VMEM((CHP, CO), jnp.float32),
            pltpu.SemaphoreType.DMA,
        ],
    )
    def body(pt_hbm, idx_hbm, mx_hbm, s1_hbm, s2_hbm,
             idx_v, rows_v, mx_v, s1_v, s2_v, sem):
        wid = lax.axis_index("s") * 2 + lax.axis_index("c")
        base_pt = wid * PPW

        def chunk_body(ci, _):
            p0 = base_pt + ci * CHP
            pltpu.sync_copy(idx_hbm.at[pl.ds(p0 * KK, CHP * KK)], idx_v)
            pltpu.async_copy(pt_hbm.at[idx_v], rows_v, sem).wait()

            def point_body(p, _):
                def col_body(c, _):
                    base = p * KK
                    cs = c * 16
                    v = rows_v[base, pl.ds(cs, 16)]
                    m = v
                    s = v
                    q = v * v
                    for r in range(1, KK):
                        v = rows_v[base + r, pl.ds(cs, 16)]
                        m = jnp.maximum(m, v)
                        s = s + v
                        q = q + v * v
                    mx_v[p, pl.ds(cs, 16)] = m
                    s1_v[p, pl.ds(cs, 16)] = s
                    s2_v[p, pl.ds(cs, 16)] = q
                    return 0

                lax.fori_loop(0, CO // 16, col_body, 0)
                return 0

            lax.fori_loop(0, CHP, point_body, 0)
            pltpu.sync_copy(mx_v, mx_hbm.at[pl.ds(p0, CHP)])
            pltpu.sync_copy(s1_v, s1_hbm.at[pl.ds(p0, CHP)])
            pltpu.sync_copy(s2_v, s2_hbm.at[pl.ds(p0, CHP)])
            return 0

        lax.fori_loop(0, PPW // CHP, chunk_body, 0)

    return body(pt_flat, idx_flat)


_RED_T = 512
_NSTEPS = BB * NN // _RED_T
_M = float(BB * NN * KK)


def _stats_body(s1_ref, s2_ref, qt_ref, out_ref, acc_ref):
    step = pl.program_id(0)

    @pl.when(step == 0)
    def _init():
        acc_ref[...] = jnp.zeros_like(acc_ref)

    s1 = s1_ref[...]
    s2 = s2_ref[...]
    qt = qt_ref[...]
    sum_y = jnp.sum(s1 + KK * qt, axis=0, keepdims=True)
    sum_y2 = jnp.sum(s2 + 2.0 * s1 * qt + KK * qt * qt, axis=0,
                     keepdims=True)
    acc_ref[0:1, :] += sum_y
    acc_ref[1:2, :] += sum_y2

    @pl.when(step == _NSTEPS - 1)
    def _fin():
        mean = acc_ref[0:1, :] / _M
        var = acc_ref[1:2, :] / _M - mean * mean
        istd = lax.rsqrt(var + 1e-5)
        out_ref[0:1, :] = mean
        out_ref[1:2, :] = istd


def _stats(s1, s2, qt_flat):
    return pl.pallas_call(
        _stats_body,
        grid=(_NSTEPS,),
        in_specs=[
            pl.BlockSpec((_RED_T, CO), lambda i: (i, 0)),
            pl.BlockSpec((_RED_T, CO), lambda i: (i, 0)),
            pl.BlockSpec((_RED_T, CO), lambda i: (i, 0)),
        ],
        out_specs=pl.BlockSpec((2, CO), lambda i: (0, 0)),
        out_shape=jax.ShapeDtypeStruct((2, CO), jnp.float32),
        scratch_shapes=[pltpu.VMEM((2, CO), jnp.float32)],
    )(s1, s2, qt_flat)


def _fin_body(m_ref, qt_ref, st_ref, g_ref, be_ref, w1_ref, b1_ref,
              w2_ref, b2_ref, out_ref):
    m = m_ref[0]
    qt = qt_ref[0]
    mean = st_ref[0:1, :]
    istd = st_ref[1:2, :]
    y = ((m + qt) - mean) * istd * g_ref[...] + be_ref[...]
    y = jnp.maximum(y, 0.0)
    w = jnp.sum(y, axis=0, keepdims=True) * (1.0 / NN)
    h = lax.dot_general(w, w1_ref[...], (((1,), (1,)), ((), ())),
                        preferred_element_type=jnp.float32)
    h = jnp.maximum(h + b1_ref[...], 0.0)
    s = lax.dot_general(h, w2_ref[...], (((1,), (1,)), ((), ())),
                        preferred_element_type=jnp.float32)
    s = jax.nn.sigmoid(s + b2_ref[...])
    out_ref[0] = jnp.transpose(y) * jnp.transpose(s)


def _finalize(mx, qt, stats, gamma2, beta2, W1, b12, W2, b22):
    hid = W1.shape[0]
    return pl.pallas_call(
        _fin_body,
        grid=(BB,),
        in_specs=[
            pl.BlockSpec((1, NN, CO), lambda b: (b, 0, 0)),
            pl.BlockSpec((1, NN, CO), lambda b: (b, 0, 0)),
            pl.BlockSpec((2, CO), lambda b: (0, 0)),
            pl.BlockSpec((1, CO), lambda b: (0, 0)),
            pl.BlockSpec((1, CO), lambda b: (0, 0)),
            pl.BlockSpec((hid, CO), lambda b: (0, 0)),
            pl.BlockSpec((1, hid), lambda b: (0, 0)),
            pl.BlockSpec((CO, hid), lambda b: (0, 0)),
            pl.BlockSpec((1, CO), lambda b: (0, 0)),
        ],
        out_specs=pl.BlockSpec((1, CO, NN), lambda b: (b, 0, 0)),
        out_shape=jax.ShapeDtypeStruct((BB, CO, NN), jnp.float32),
    )(mx, qt, stats, gamma2, beta2, W1, b12, W2, b22)


def kernel(x, W_conv, gamma, beta, W1, b1, W2, b2):
    wa = W_conv[:, :CC]
    wd = W_conv[:, CC:] - wa

    pt, qt, sq = _prep(x, wa, wd)
    idx = _knn(x, sq)
    mx, s1, s2 = _sc_gather_reduce(
        pt.reshape(BB * NN, CO), idx.reshape(BB * NN * KK))
    stats = _stats(s1, s2, qt.reshape(BB * NN, CO))
    out = _finalize(
        mx.reshape(BB, NN, CO), qt, stats,
        gamma.reshape(1, CO), beta.reshape(1, CO),
        W1, b1.reshape(1, -1), W2, b2.reshape(1, CO))
    return out


# speedup vs baseline: 9.8214x; 1.4230x over previous
import functools

import jax
import jax.numpy as jnp
from jax import lax
from jax.experimental import pallas as pl
from jax.experimental.pallas import tpu as pltpu
from jax.experimental.pallas import tpu_sc as plsc

BB, CC, NN, KK = 8, 256, 2048, 16
CO = 256
TN = 256
NW = 32
PPW = BB * NN // NW
CHP = 8


def _prep_body(x_ref, wa_ref, wd_ref, pt_ref, qt_ref, sq_ref):
    xb = x_ref[0]
    pt_ref[0] = lax.dot_general(xb, wa_ref[...], (((0,), (1,)), ((), ())),
                                preferred_element_type=jnp.float32)
    qt_ref[0] = lax.dot_general(xb, wd_ref[...], (((0,), (1,)), ((), ())),
                                preferred_element_type=jnp.float32)
    sq_ref[0, 0] = jnp.sum(xb * xb, axis=0)


def _prep(x, wa, wd):
    return pl.pallas_call(
        _prep_body,
        grid=(BB,),
        in_specs=[
            pl.BlockSpec((1, CC, NN), lambda b: (b, 0, 0)),
            pl.BlockSpec((CO, CC), lambda b: (0, 0)),
            pl.BlockSpec((CO, CC), lambda b: (0, 0)),
        ],
        out_specs=[
            pl.BlockSpec((1, NN, CO), lambda b: (b, 0, 0)),
            pl.BlockSpec((1, NN, CO), lambda b: (b, 0, 0)),
            pl.BlockSpec((1, 1, NN), lambda b: (b, 0, 0)),
        ],
        out_shape=[
            jax.ShapeDtypeStruct((BB, NN, CO), jnp.float32),
            jax.ShapeDtypeStruct((BB, NN, CO), jnp.float32),
            jax.ShapeDtypeStruct((BB, 1, NN), jnp.float32),
        ],
    )(x, wa, wd)


def _knn_body(xrow_ref, xall_ref, sq_ref, sqrow_ref, idx_ref):
    b = pl.program_id(0)
    xr = xrow_ref[0]
    xa = xall_ref[0]
    inner = lax.dot_general(xr, xa, (((0,), (0,)), ((), ())),
                            preferred_element_type=jnp.float32)
    sq = sq_ref[0]
    sqr = sqrow_ref[0]
    d = jnp.transpose(sqr) + sq - 2.0 * inner
    col_ids = lax.broadcasted_iota(jnp.int32, (TN, NN), 1)
    cols = []
    for _ in range(KK):
        am = jnp.argmin(d, axis=1).astype(jnp.int32).reshape(TN, 1)
        cols.append(am)
        d = jnp.where(col_ids == am, jnp.inf, d)
    idx_ref[0] = jnp.concatenate(cols, axis=1) + b * NN


def _knn(x, sq):
    return pl.pallas_call(
        _knn_body,
        grid=(BB, NN // TN),
        in_specs=[
            pl.BlockSpec((1, CC, TN), lambda b, j: (b, 0, j)),
            pl.BlockSpec((1, CC, NN), lambda b, j: (b, 0, 0)),
            pl.BlockSpec((1, 1, NN), lambda b, j: (b, 0, 0)),
            pl.BlockSpec((1, 1, TN), lambda b, j: (b, 0, j)),
        ],
        out_specs=pl.BlockSpec((1, TN, KK), lambda b, j: (b, j, 0)),
        out_shape=jax.ShapeDtypeStruct((BB, NN, KK), jnp.int32),
    )(x, x, sq, sq)


_NCH = PPW // CHP
_RPC = CHP * KK


def _sc_gather_reduce(pt_flat, idx2d):
    mesh = plsc.VectorSubcoreMesh(core_axis_name="c", subcore_axis_name="s")

    @functools.partial(
        pl.kernel,
        mesh=mesh,
        out_type=(
            jax.ShapeDtypeStruct((BB * NN, CO), jnp.float32),
            jax.ShapeDtypeStruct((BB * NN, CO), jnp.float32),
            jax.ShapeDtypeStruct((BB * NN, CO), jnp.float32),
        ),
        scratch_types=[
            pltpu.VMEM((_NCH, _RPC), jnp.int32),
            pltpu.VMEM((_RPC, CO), jnp.float32),
            pltpu.VMEM((_RPC, CO), jnp.float32),
            pltpu.VMEM((CHP, CO), jnp.float32),
            pltpu.VMEM((CHP, CO), jnp.float32),
            pltpu.VMEM((CHP, CO), jnp.float32),
            pltpu.VMEM((CHP, CO), jnp.float32),
            pltpu.VMEM((CHP, CO), jnp.float32),
            pltpu.VMEM((CHP, CO), jnp.float32),
            pltpu.SemaphoreType.DMA,
            pltpu.SemaphoreType.DMA,
            pltpu.SemaphoreType.DMA,
            pltpu.SemaphoreType.DMA,
        ],
    )
    def body(pt_hbm, idx_hbm, mx_hbm, s1_hbm, s2_hbm,
             idx_v, rows_v0, rows_v1, mx_v0, mx_v1, s1_v0, s1_v1,
             s2_v0, s2_v1, gsem0, gsem1, osem0, osem1):
        wid = lax.axis_index("s") * 2 + lax.axis_index("c")
        base_pt = wid * PPW
        rows_b = (rows_v0, rows_v1)
        mx_b = (mx_v0, mx_v1)
        s1_b = (s1_v0, s1_v1)
        s2_b = (s2_v0, s2_v1)
        gsem = (gsem0, gsem1)
        osem = (osem0, osem1)

        pltpu.sync_copy(idx_hbm.at[pl.ds(wid * _NCH, _NCH)], idx_v)

        def start_gather(ci, par):
            pltpu.make_async_copy(
                pt_hbm.at[idx_v.at[ci]], rows_b[par], gsem[par]).start()

        def wait_gather(ci, par):
            pltpu.make_async_copy(
                pt_hbm.at[idx_v.at[ci]], rows_b[par], gsem[par]).wait()

        def compute_chunk(ci, par):
            rows_v = rows_b[par]
            mx_v, s1_v, s2_v = mx_b[par], s1_b[par], s2_b[par]

            def point_body(p, _):
                def col_body(c, _):
                    base = p * KK
                    cs = c * 16
                    v = rows_v[base, pl.ds(cs, 16)]
                    m = v
                    s = v
                    q = v * v
                    for r in range(1, KK):
                        v = rows_v[base + r, pl.ds(cs, 16)]
                        m = jnp.maximum(m, v)
                        s = s + v
                        q = q + v * v
                    mx_v[p, pl.ds(cs, 16)] = m
                    s1_v[p, pl.ds(cs, 16)] = s
                    s2_v[p, pl.ds(cs, 16)] = q
                    return 0

                lax.fori_loop(0, CO // 16, col_body, 0)
                return 0

            lax.fori_loop(0, CHP, point_body, 0)
            p0 = base_pt + ci * CHP
            pltpu.make_async_copy(mx_v, mx_hbm.at[pl.ds(p0, CHP)],
                                  osem[par]).start()
            pltpu.make_async_copy(s1_v, s1_hbm.at[pl.ds(p0, CHP)],
                                  osem[par]).start()
            pltpu.make_async_copy(s2_v, s2_hbm.at[pl.ds(p0, CHP)],
                                  osem[par]).start()

        def wait_out(ci, par):
            pltpu.make_async_copy(mx_b[par], mx_hbm.at[pl.ds(0, CHP)],
                                  osem[par]).wait()
            pltpu.make_async_copy(s1_b[par], s1_hbm.at[pl.ds(0, CHP)],
                                  osem[par]).wait()
            pltpu.make_async_copy(s2_b[par], s2_hbm.at[pl.ds(0, CHP)],
                                  osem[par]).wait()

        start_gather(0, 0)

        def pair_body(pi, _):
            for q in range(2):
                ci = pi * 2 + q
                par = q
                @pl.when(ci + 1 < _NCH)
                def _qnext():
                    start_gather(ci + 1, 1 - par)

                wait_gather(ci, par)

                @pl.when(ci >= 2)
                def _drain():
                    wait_out(ci - 2, par)

                compute_chunk(ci, par)
            return 0

        lax.fori_loop(0, _NCH // 2, pair_body, 0)
        wait_out(_NCH - 2, 0)
        wait_out(_NCH - 1, 1)

    return body(pt_flat, idx2d)


_RED_T = 512
_NSTEPS = BB * NN // _RED_T
_M = float(BB * NN * KK)


def _stats_body(s1_ref, s2_ref, qt_ref, out_ref, acc_ref):
    step = pl.program_id(0)

    @pl.when(step == 0)
    def _init():
        acc_ref[...] = jnp.zeros_like(acc_ref)

    s1 = s1_ref[...]
    s2 = s2_ref[...]
    qt = qt_ref[...]
    sum_y = jnp.sum(s1 + KK * qt, axis=0, keepdims=True)
    sum_y2 = jnp.sum(s2 + 2.0 * s1 * qt + KK * qt * qt, axis=0,
                     keepdims=True)
    acc_ref[0:1, :] += sum_y
    acc_ref[1:2, :] += sum_y2

    @pl.when(step == _NSTEPS - 1)
    def _fin():
        mean = acc_ref[0:1, :] / _M
        var = acc_ref[1:2, :] / _M - mean * mean
        istd = lax.rsqrt(var + 1e-5)
        out_ref[0:1, :] = mean
        out_ref[1:2, :] = istd


def _stats(s1, s2, qt_flat):
    return pl.pallas_call(
        _stats_body,
        grid=(_NSTEPS,),
        in_specs=[
            pl.BlockSpec((_RED_T, CO), lambda i: (i, 0)),
            pl.BlockSpec((_RED_T, CO), lambda i: (i, 0)),
            pl.BlockSpec((_RED_T, CO), lambda i: (i, 0)),
        ],
        out_specs=pl.BlockSpec((2, CO), lambda i: (0, 0)),
        out_shape=jax.ShapeDtypeStruct((2, CO), jnp.float32),
        scratch_shapes=[pltpu.VMEM((2, CO), jnp.float32)],
    )(s1, s2, qt_flat)


def _fin_body(m_ref, qt_ref, st_ref, g_ref, be_ref, w1_ref, b1_ref,
              w2_ref, b2_ref, out_ref):
    m = m_ref[0]
    qt = qt_ref[0]
    mean = st_ref[0:1, :]
    istd = st_ref[1:2, :]
    y = ((m + qt) - mean) * istd * g_ref[...] + be_ref[...]
    y = jnp.maximum(y, 0.0)
    w = jnp.sum(y, axis=0, keepdims=True) * (1.0 / NN)
    h = lax.dot_general(w, w1_ref[...], (((1,), (1,)), ((), ())),
                        preferred_element_type=jnp.float32)
    h = jnp.maximum(h + b1_ref[...], 0.0)
    s = lax.dot_general(h, w2_ref[...], (((1,), (1,)), ((), ())),
                        preferred_element_type=jnp.float32)
    s = jax.nn.sigmoid(s + b2_ref[...])
    out_ref[0] = jnp.transpose(y) * jnp.transpose(s)


def _finalize(mx, qt, stats, gamma2, beta2, W1, b12, W2, b22):
    hid = W1.shape[0]
    return pl.pallas_call(
        _fin_body,
        grid=(BB,),
        in_specs=[
            pl.BlockSpec((1, NN, CO), lambda b: (b, 0, 0)),
            pl.BlockSpec((1, NN, CO), lambda b: (b, 0, 0)),
            pl.BlockSpec((2, CO), lambda b: (0, 0)),
            pl.BlockSpec((1, CO), lambda b: (0, 0)),
            pl.BlockSpec((1, CO), lambda b: (0, 0)),
            pl.BlockSpec((hid, CO), lambda b: (0, 0)),
            pl.BlockSpec((1, hid), lambda b: (0, 0)),
            pl.BlockSpec((CO, hid), lambda b: (0, 0)),
            pl.BlockSpec((1, CO), lambda b: (0, 0)),
        ],
        out_specs=pl.BlockSpec((1, CO, NN), lambda b: (b, 0, 0)),
        out_shape=jax.ShapeDtypeStruct((BB, CO, NN), jnp.float32),
    )(mx, qt, stats, gamma2, beta2, W1, b12, W2, b22)


def kernel(x, W_conv, gamma, beta, W1, b1, W2, b2):
    wa = W_conv[:, :CC]
    wd = W_conv[:, CC:] - wa

    pt, qt, sq = _prep(x, wa, wd)
    idx = _knn(x, sq)
    mx, s1, s2 = _sc_gather_reduce(
        pt.reshape(BB * NN, CO), idx.reshape(BB * NN * KK // _RPC, _RPC))
    stats = _stats(s1, s2, qt.reshape(BB * NN, CO))
    out = _finalize(
        mx.reshape(BB, NN, CO), qt, stats,
        gamma.reshape(1, CO), beta.reshape(1, CO),
        W1, b1.reshape(1, -1), W2, b2.reshape(1, CO))
    return out


# speedup vs baseline: 10.8783x; 1.1076x over previous
import functools

import jax
import jax.numpy as jnp
from jax import lax
from jax.experimental import pallas as pl
from jax.experimental.pallas import tpu as pltpu
from jax.experimental.pallas import tpu_sc as plsc

BB, CC, NN, KK = 8, 256, 2048, 16
CO = 256
TN = 256
NW = 32
CHP = 8
_RPC = CHP * KK
SPLITS = (4, 2, 2)
_MTOT = float(BB * NN * KK)


def _prep_body(x_ref, wa_ref, wd_ref, pt_ref, qt_ref, sq_ref):
    xb = x_ref[0]
    pt_ref[0] = lax.dot_general(xb, wa_ref[...], (((0,), (1,)), ((), ())),
                                preferred_element_type=jnp.float32)
    qt_ref[0] = lax.dot_general(xb, wd_ref[...], (((0,), (1,)), ((), ())),
                                preferred_element_type=jnp.float32)
    sq_ref[0, 0] = jnp.sum(xb * xb, axis=0)


def _prep(x, wa, wd):
    return pl.pallas_call(
        _prep_body,
        grid=(BB,),
        in_specs=[
            pl.BlockSpec((1, CC, NN), lambda b: (b, 0, 0)),
            pl.BlockSpec((CO, CC), lambda b: (0, 0)),
            pl.BlockSpec((CO, CC), lambda b: (0, 0)),
        ],
        out_specs=[
            pl.BlockSpec((1, NN, CO), lambda b: (b, 0, 0)),
            pl.BlockSpec((1, NN, CO), lambda b: (b, 0, 0)),
            pl.BlockSpec((1, 1, NN), lambda b: (b, 0, 0)),
        ],
        out_shape=[
            jax.ShapeDtypeStruct((BB, NN, CO), jnp.float32),
            jax.ShapeDtypeStruct((BB, NN, CO), jnp.float32),
            jax.ShapeDtypeStruct((BB, 1, NN), jnp.float32),
        ],
    )(x, wa, wd)


def _knn_body(off, xrow_ref, xall_ref, sq_ref, sqrow_ref, idx_ref):
    b = pl.program_id(0)
    xr = xrow_ref[0]
    xa = xall_ref[0]
    inner = lax.dot_general(xr, xa, (((0,), (0,)), ((), ())),
                            preferred_element_type=jnp.float32)
    sq = sq_ref[0]
    sqr = sqrow_ref[0]
    d = jnp.transpose(sqr) + sq - 2.0 * inner
    col_ids = lax.broadcasted_iota(jnp.int32, (TN, NN), 1)
    cols = []
    for _ in range(KK):
        am = jnp.argmin(d, axis=1).astype(jnp.int32).reshape(TN, 1)
        cols.append(am)
        d = jnp.where(col_ids == am, jnp.inf, d)
    idx_ref[0] = jnp.concatenate(cols, axis=1) + (b + off) * NN


def _knn(x, sq, off, nb):
    return pl.pallas_call(
        functools.partial(_knn_body, off),
        grid=(nb, NN // TN),
        in_specs=[
            pl.BlockSpec((1, CC, TN), lambda b, j: (b + off, 0, j)),
            pl.BlockSpec((1, CC, NN), lambda b, j: (b + off, 0, 0)),
            pl.BlockSpec((1, 1, NN), lambda b, j: (b + off, 0, 0)),
            pl.BlockSpec((1, 1, TN), lambda b, j: (b + off, 0, j)),
        ],
        out_specs=pl.BlockSpec((1, TN, KK), lambda b, j: (b, j, 0)),
        out_shape=jax.ShapeDtypeStruct((nb, NN, KK), jnp.int32),
    )(x, x, sq, sq)


def _sc_gather_reduce(pt_flat, idx2d, nb):
    ppw = nb * NN // NW
    nch = ppw // CHP
    mesh = plsc.VectorSubcoreMesh(core_axis_name="c", subcore_axis_name="s")

    @functools.partial(
        pl.kernel,
        mesh=mesh,
        out_type=(
            jax.ShapeDtypeStruct((nb * NN, CO), jnp.float32),
            jax.ShapeDtypeStruct((nb * NN, CO), jnp.float32),
            jax.ShapeDtypeStruct((nb * NN, CO), jnp.float32),
        ),
        scratch_types=[
            pltpu.VMEM((nch, _RPC), jnp.int32),
            pltpu.VMEM((_RPC, CO), jnp.float32),
            pltpu.VMEM((_RPC, CO), jnp.float32),
            pltpu.VMEM((CHP, CO), jnp.float32),
            pltpu.VMEM((CHP, CO), jnp.float32),
            pltpu.VMEM((CHP, CO), jnp.float32),
            pltpu.VMEM((CHP, CO), jnp.float32),
            pltpu.VMEM((CHP, CO), jnp.float32),
            pltpu.VMEM((CHP, CO), jnp.float32),
            pltpu.SemaphoreType.DMA,
            pltpu.SemaphoreType.DMA,
            pltpu.SemaphoreType.DMA,
            pltpu.SemaphoreType.DMA,
        ],
    )
    def body(pt_hbm, idx_hbm, mx_hbm, s1_hbm, s2_hbm,
             idx_v, rows_v0, rows_v1, mx_v0, mx_v1, s1_v0, s1_v1,
             s2_v0, s2_v1, gsem0, gsem1, osem0, osem1):
        wid = lax.axis_index("s") * 2 + lax.axis_index("c")
        base_pt = wid * ppw
        rows_b = (rows_v0, rows_v1)
        mx_b = (mx_v0, mx_v1)
        s1_b = (s1_v0, s1_v1)
        s2_b = (s2_v0, s2_v1)
        gsem = (gsem0, gsem1)
        osem = (osem0, osem1)

        pltpu.sync_copy(idx_hbm.at[pl.ds(wid * nch, nch)], idx_v)

        def start_gather(ci, par):
            pltpu.make_async_copy(
                pt_hbm.at[idx_v.at[ci]], rows_b[par], gsem[par]).start()

        def wait_gather(ci, par):
            pltpu.make_async_copy(
                pt_hbm.at[idx_v.at[ci]], rows_b[par], gsem[par]).wait()

        def compute_chunk(ci, par):
            rows_v = rows_b[par]
            mx_v, s1_v, s2_v = mx_b[par], s1_b[par], s2_b[par]

            def point_body(p, _):
                def col_body(c, _):
                    base = p * KK
                    cs = c * 16
                    v = rows_v[base, pl.ds(cs, 16)]
                    m = v
                    s = v
                    q = v * v
                    for r in range(1, KK):
                        v = rows_v[base + r, pl.ds(cs, 16)]
                        m = jnp.maximum(m, v)
                        s = s + v
                        q = q + v * v
                    mx_v[p, pl.ds(cs, 16)] = m
                    s1_v[p, pl.ds(cs, 16)] = s
                    s2_v[p, pl.ds(cs, 16)] = q
                    return 0

                lax.fori_loop(0, CO // 16, col_body, 0)
                return 0

            lax.fori_loop(0, CHP, point_body, 0)
            p0 = base_pt + ci * CHP
            pltpu.make_async_copy(mx_v, mx_hbm.at[pl.ds(p0, CHP)],
                                  osem[par]).start()
            pltpu.make_async_copy(s1_v, s1_hbm.at[pl.ds(p0, CHP)],
                                  osem[par]).start()
            pltpu.make_async_copy(s2_v, s2_hbm.at[pl.ds(p0, CHP)],
                                  osem[par]).start()

        def wait_out(par):
            pltpu.make_async_copy(mx_b[par], mx_hbm.at[pl.ds(0, CHP)],
                                  osem[par]).wait()
            pltpu.make_async_copy(s1_b[par], s1_hbm.at[pl.ds(0, CHP)],
                                  osem[par]).wait()
            pltpu.make_async_copy(s2_b[par], s2_hbm.at[pl.ds(0, CHP)],
                                  osem[par]).wait()

        start_gather(0, 0)

        def pair_body(pi, _):
            for q in range(2):
                ci = pi * 2 + q
                par = q

                @pl.when(ci + 1 < nch)
                def _qnext():
                    start_gather(ci + 1, 1 - par)

                wait_gather(ci, par)

                @pl.when(ci >= 2)
                def _drain():
                    wait_out(par)

                compute_chunk(ci, par)
            return 0

        lax.fori_loop(0, nch // 2, pair_body, 0)
        wait_out(0)
        wait_out(1)

    return body(pt_flat, idx2d)


_RED_T = 512


def _stats_body(nsteps, s1_ref, s2_ref, qt_ref, out_ref, acc_ref):
    step = pl.program_id(0)

    @pl.when(step == 0)
    def _init():
        acc_ref[...] = jnp.zeros_like(acc_ref)

    s1 = s1_ref[...]
    s2 = s2_ref[...]
    qt = qt_ref[0]
    sum_y = jnp.sum(s1 + KK * qt, axis=0, keepdims=True)
    sum_y2 = jnp.sum(s2 + 2.0 * s1 * qt + KK * qt * qt, axis=0,
                     keepdims=True)
    acc_ref[0:1, :] += sum_y
    acc_ref[1:2, :] += sum_y2

    @pl.when(step == nsteps - 1)
    def _fin():
        out_ref[...] = acc_ref[...]


def _stats(s1, s2, qt3, off, nb):
    nsteps = nb * NN // _RED_T
    soff = off * NN // _RED_T
    return pl.pallas_call(
        functools.partial(_stats_body, nsteps),
        grid=(nsteps,),
        in_specs=[
            pl.BlockSpec((_RED_T, CO), lambda i: (i, 0)),
            pl.BlockSpec((_RED_T, CO), lambda i: (i, 0)),
            pl.BlockSpec((1, _RED_T, CO), lambda i: (i + soff, 0, 0)),
        ],
        out_specs=pl.BlockSpec((2, CO), lambda i: (0, 0)),
        out_shape=jax.ShapeDtypeStruct((2, CO), jnp.float32),
        scratch_shapes=[pltpu.VMEM((2, CO), jnp.float32)],
    )(s1, s2, qt3)


def _fin_body(nsums, m_ref, qt_ref, g_ref, be_ref, w1_ref, b1_ref,
              w2_ref, b2_ref, *rest):
    sum_refs = rest[:nsums]
    out_ref = rest[nsums]
    acc = sum_refs[0][...]
    for r in sum_refs[1:]:
        acc = acc + r[...]
    mean = acc[0:1, :] / _MTOT
    var = acc[1:2, :] / _MTOT - mean * mean
    istd = lax.rsqrt(var + 1e-5)
    m = m_ref[0]
    qt = qt_ref[0]
    y = ((m + qt) - mean) * istd * g_ref[...] + be_ref[...]
    y = jnp.maximum(y, 0.0)
    w = jnp.sum(y, axis=0, keepdims=True) * (1.0 / NN)
    h = lax.dot_general(w, w1_ref[...], (((1,), (1,)), ((), ())),
                        preferred_element_type=jnp.float32)
    h = jnp.maximum(h + b1_ref[...], 0.0)
    s = lax.dot_general(h, w2_ref[...], (((1,), (1,)), ((), ())),
                        preferred_element_type=jnp.float32)
    s = jax.nn.sigmoid(s + b2_ref[...])
    out_ref[0] = jnp.transpose(y) * jnp.transpose(s)


def _finalize(mx, qt, sums, gamma2, beta2, W1, b12, W2, b22, off, nb):
    hid = W1.shape[0]
    nsums = len(sums)
    full = lambda b: (0, 0)
    return pl.pallas_call(
        functools.partial(_fin_body, nsums),
        grid=(nb,),
        in_specs=[
            pl.BlockSpec((1, NN, CO), lambda b: (b, 0, 0)),
            pl.BlockSpec((1, NN, CO), lambda b: (b + off, 0, 0)),
            pl.BlockSpec((1, CO), full),
            pl.BlockSpec((1, CO), full),
            pl.BlockSpec((hid, CO), full),
            pl.BlockSpec((1, hid), full),
            pl.BlockSpec((CO, hid), full),
            pl.BlockSpec((1, CO), full),
        ] + [pl.BlockSpec((2, CO), full) for _ in range(nsums)],
        out_specs=pl.BlockSpec((1, CO, NN), lambda b: (b, 0, 0)),
        out_shape=jax.ShapeDtypeStruct((nb, CO, NN), jnp.float32),
    )(mx, qt, gamma2, beta2, W1, b12, W2, b22, *sums)


def kernel(x, W_conv, gamma, beta, W1, b1, W2, b2):
    wa = W_conv[:, :CC]
    wd = W_conv[:, CC:] - wa

    pt, qt, sq = _prep(x, wa, wd)
    pt_flat = pt.reshape(BB * NN, CO)
    qt3 = qt.reshape(BB * NN // _RED_T, _RED_T, CO)

    mxs, sums = [], []
    off = 0
    for nb in SPLITS:
        idx = _knn(x, sq, off, nb)
        mx, s1, s2 = _sc_gather_reduce(
            pt_flat, idx.reshape(nb * NN * KK // _RPC, _RPC), nb)
        sums.append(_stats(s1, s2, qt3, off, nb))
        mxs.append(mx)
        off += nb

    outs = []
    off = 0
    for mx, nb in zip(mxs, SPLITS):
        outs.append(_finalize(
            mx.reshape(nb, NN, CO), qt, sums,
            gamma.reshape(1, CO), beta.reshape(1, CO),
            W1, b1.reshape(1, -1), W2, b2.reshape(1, CO), off, nb))
        off += nb
    return jnp.concatenate(outs, axis=0)
